# baseline (speedup 1.0000x reference)
import jax
import jax.numpy as jnp
from jax.experimental import pallas as pl
from jax.experimental.pallas import tpu as pltpu

B, N, D = 4, 128, 128
HID, REL = 256, 64
SUB = 16


def _fused_body(p_ref, pbf_ref, a_ref, w1_ref, b1_ref, w2_ref, b2_ref,
                w3_ref, b3_ref, w4_ref, b4_ref, out_ref):
    p_bf = pbf_ref[...]
    s_all = (jnp.dot(p_bf, w1_ref[:D, :], preferred_element_type=jnp.float32)
             + b1_ref[0][None, :]).astype(jnp.bfloat16)
    r_all = jnp.dot(p_bf, w1_ref[D:, :],
                    preferred_element_type=jnp.float32).astype(jnp.bfloat16)

    b2v = b2_ref[0][None, :]
    rel_blocks = []
    for b in range(B):
        r_b = r_all[b * N:(b + 1) * N, :]
        contribs = []
        for u in range(N // SUB):
            lo = b * N + u * SUB
            s_blk = s_all[lo:lo + SUB, :]
            h = jnp.maximum(s_blk[:, None, :] + r_b[None, :, :],
                            jnp.bfloat16(0.0))
            f = jnp.dot(h.reshape(SUB * N, HID), w2_ref[...],
                        preferred_element_type=jnp.float32
                        ).astype(jnp.bfloat16) + b2v
            f = jnp.maximum(f, jnp.bfloat16(0.0)).reshape(SUB, N, REL)
            a_blk = a_ref[u * SUB:(u + 1) * SUB, :]
            contribs.append(jnp.sum(f * a_blk[:, :, None], axis=0))
        rel_blocks.append(sum(contribs))
    rel = jnp.concatenate(rel_blocks, axis=0)

    h2 = jnp.maximum(
        jnp.dot(p_bf, w3_ref[:D, :], preferred_element_type=jnp.float32)
        + jnp.dot(rel, w3_ref[D:, :], preferred_element_type=jnp.float32)
        + b3_ref[0][None, :],
        0.0).astype(jnp.bfloat16)
    delta = jnp.dot(h2, w4_ref[...], preferred_element_type=jnp.float32) \
        + b4_ref[0][None, :]
    out_ref[...] = p_ref[...] + delta


def kernel(particles, adjacency_matrix, W1, b1, W2, b2, W3, b3, W4, b4):
    p2 = particles.reshape(B * N, D)
    mask = (adjacency_matrix == 1).astype(jnp.bfloat16)
    out = pl.pallas_call(
        _fused_body,
        out_shape=jax.ShapeDtypeStruct((B * N, D), jnp.float32),
    )(p2, p2.astype(jnp.bfloat16), mask,
      W1.astype(jnp.bfloat16), b1.reshape(1, HID),
      W2.astype(jnp.bfloat16), b2.reshape(1, REL).astype(jnp.bfloat16),
      W3.astype(jnp.bfloat16), b3.reshape(1, HID),
      W4.astype(jnp.bfloat16), b4.reshape(1, D))
    return out.reshape(B, N, D)


# speedup vs baseline: 1.0658x; 1.0658x over previous
import jax
import jax.numpy as jnp
from jax.experimental import pallas as pl
from jax.experimental.pallas import tpu as pltpu

B, N, D = 4, 128, 128
HID, REL = 256, 64
SUB = 16


def _fused_body(p_ref, pbf_ref, a_ref, w1_ref, b1_ref, w2_ref, b2_ref,
                w3_ref, b3_ref, w4_ref, b4_ref, out_ref):
    p_bf = pbf_ref[...]
    s_all = (jnp.dot(p_bf, w1_ref[:D, :], preferred_element_type=jnp.float32)
             + b1_ref[0][None, :]).astype(jnp.bfloat16)
    r_all = jnp.dot(p_bf, w1_ref[D:, :],
                    preferred_element_type=jnp.float32).astype(jnp.bfloat16)

    b2v = b2_ref[0][None, :]
    rel_blocks = []
    for b in range(B):
        r_b = r_all[b * N:(b + 1) * N, :]
        contribs = []
        for u in range(N // SUB):
            lo = b * N + u * SUB
            s_blk = s_all[lo:lo + SUB, :]
            h = jnp.maximum(s_blk[:, None, :] + r_b[None, :, :],
                            jnp.bfloat16(0.0))
            f = jnp.dot(h.reshape(SUB * N, HID), w2_ref[...],
                        preferred_element_type=jnp.float32) + b2v
            f = jnp.maximum(f, 0.0).reshape(SUB, N, REL)
            a_blk = a_ref[u * SUB:(u + 1) * SUB, :]
            contribs.append(jnp.sum(f * a_blk[:, :, None], axis=0))
        rel_blocks.append(sum(contribs))
    rel = jnp.concatenate(rel_blocks, axis=0).astype(jnp.bfloat16)

    h2 = jnp.maximum(
        jnp.dot(p_bf, w3_ref[:D, :], preferred_element_type=jnp.float32)
        + jnp.dot(rel, w3_ref[D:, :], preferred_element_type=jnp.float32)
        + b3_ref[0][None, :],
        0.0).astype(jnp.bfloat16)
    delta = jnp.dot(h2, w4_ref[...], preferred_element_type=jnp.float32) \
        + b4_ref[0][None, :]
    out_ref[...] = p_ref[...] + delta


def kernel(particles, adjacency_matrix, W1, b1, W2, b2, W3, b3, W4, b4):
    p2 = particles.reshape(B * N, D)
    mask = (adjacency_matrix == 1).astype(jnp.bfloat16)
    out = pl.pallas_call(
        _fused_body,
        out_shape=jax.ShapeDtypeStruct((B * N, D), jnp.float32),
    )(p2, p2.astype(jnp.bfloat16), mask,
      W1.astype(jnp.bfloat16), b1.reshape(1, HID),
      W2.astype(jnp.bfloat16), b2.reshape(1, REL),
      W3.astype(jnp.bfloat16), b3.reshape(1, HID),
      W4.astype(jnp.bfloat16), b4.reshape(1, D))
    return out.reshape(B, N, D)


# speedup vs baseline: 1.2841x; 1.2048x over previous
import jax
import jax.numpy as jnp
from jax.experimental import pallas as pl
from jax.experimental.pallas import tpu as pltpu

B, N, D = 4, 128, 128
HID, REL = 256, 64
SUB = 16


def _fused_body(p_ref, a_ref, w1_ref, b1_ref, w2_ref, b2_ref,
                w3_ref, b3_ref, w4_ref, b4_ref, out_ref):
    p_all = p_ref[...]
    s_all = (jnp.dot(p_all, w1_ref[:D, :], preferred_element_type=jnp.float32)
             + b1_ref[0][None, :]).astype(jnp.bfloat16)
    r_all = jnp.dot(p_all, w1_ref[D:, :],
                    preferred_element_type=jnp.float32).astype(jnp.bfloat16)

    rel_blocks = []
    for b in range(B):
        r_b = r_all[b * N:(b + 1) * N, :]
        contribs = []
        for u in range(N // SUB):
            lo = b * N + u * SUB
            s_blk = s_all[lo:lo + SUB, :]
            h = jnp.maximum(s_blk[:, None, :] + r_b[None, :, :],
                            jnp.bfloat16(0.0))
            f = jnp.dot(h.reshape(SUB * N, HID), w2_ref[...],
                        preferred_element_type=jnp.float32) + b2_ref[0][None, :]
            f = jnp.maximum(f, 0.0).reshape(SUB, N, REL)
            a_blk = a_ref[u * SUB:(u + 1) * SUB, :]
            contribs.append(jnp.sum(f * a_blk[:, :, None], axis=0))
        rel_blocks.append(sum(contribs))
    rel = jnp.concatenate(rel_blocks, axis=0)

    h2 = jnp.maximum(
        jnp.dot(p_all, w3_ref[:D, :], preferred_element_type=jnp.float32)
        + jnp.dot(rel, w3_ref[D:, :], preferred_element_type=jnp.float32)
        + b3_ref[0][None, :],
        0.0)
    delta = jnp.dot(h2, w4_ref[...], preferred_element_type=jnp.float32) \
        + b4_ref[0][None, :]
    out_ref[...] = p_all + delta


def kernel(particles, adjacency_matrix, W1, b1, W2, b2, W3, b3, W4, b4):
    mask = (adjacency_matrix == 1).astype(jnp.float32)
    out = pl.pallas_call(
        _fused_body,
        out_shape=jax.ShapeDtypeStruct((B * N, D), jnp.float32),
    )(particles.reshape(B * N, D), mask, W1, b1.reshape(1, HID),
      W2.astype(jnp.bfloat16), b2.reshape(1, REL),
      W3, b3.reshape(1, HID), W4, b4.reshape(1, D))
    return out.reshape(B, N, D)


# speedup vs baseline: 1.3348x; 1.0395x over previous
import jax
import jax.numpy as jnp
from jax.experimental import pallas as pl
from jax.experimental.pallas import tpu as pltpu

B, N, D = 4, 128, 128
HID, REL = 256, 64
SUB = 32


def _fused_body(p_ref, a_ref, w1_ref, b1_ref, w2_ref, b2_ref,
                w3_ref, b3_ref, w4_ref, b4_ref, out_ref):
    p_all = p_ref[...]
    a_mask = (a_ref[...] == 1).astype(jnp.float32)
    w2 = w2_ref[...].astype(jnp.bfloat16)
    s_all = (jnp.dot(p_all, w1_ref[:D, :], preferred_element_type=jnp.float32)
             + b1_ref[0][None, :]).astype(jnp.bfloat16)
    r_all = jnp.dot(p_all, w1_ref[D:, :],
                    preferred_element_type=jnp.float32).astype(jnp.bfloat16)

    rel_blocks = []
    for b in range(B):
        r_b = r_all[b * N:(b + 1) * N, :]
        contribs = []
        for u in range(N // SUB):
            lo = b * N + u * SUB
            s_blk = s_all[lo:lo + SUB, :]
            h = jnp.maximum(s_blk[:, None, :] + r_b[None, :, :],
                            jnp.bfloat16(0.0))
            f = jnp.dot(h.reshape(SUB * N, HID), w2,
                        preferred_element_type=jnp.float32) + b2_ref[0][None, :]
            f = jnp.maximum(f, 0.0).reshape(SUB, N, REL)
            a_blk = a_mask[u * SUB:(u + 1) * SUB, :]
            contribs.append(jnp.sum(f * a_blk[:, :, None], axis=0))
        rel_blocks.append(sum(contribs))
    rel = jnp.concatenate(rel_blocks, axis=0)

    h2 = jnp.maximum(
        jnp.dot(p_all, w3_ref[:D, :], preferred_element_type=jnp.float32)
        + jnp.dot(rel, w3_ref[D:, :], preferred_element_type=jnp.float32)
        + b3_ref[0][None, :],
        0.0)
    delta = jnp.dot(h2, w4_ref[...], preferred_element_type=jnp.float32) \
        + b4_ref[0][None, :]
    out_ref[...] = p_all + delta


def kernel(particles, adjacency_matrix, W1, b1, W2, b2, W3, b3, W4, b4):
    out = pl.pallas_call(
        _fused_body,
        out_shape=jax.ShapeDtypeStruct((B * N, D), jnp.float32),
    )(particles.reshape(B * N, D), adjacency_matrix, W1, b1.reshape(1, HID),
      W2, b2.reshape(1, REL),
      W3, b3.reshape(1, HID), W4, b4.reshape(1, D))
    return out.reshape(B, N, D)
